# Initial kernel scaffold; baseline (speedup 1.0000x reference)
import jax
import jax.numpy as jnp
from jax.experimental import pallas as pl


def kernel(x, edge_index, W_l, b_l, W_r):
    raise NotImplementedError("write your pallas kernel here")


# speedup vs baseline: 3.3117x; 3.3117x over previous
import functools

import jax
import jax.numpy as jnp
from jax import lax
from jax.experimental import pallas as pl
from jax.experimental.pallas import tpu as pltpu
from jax.experimental.pallas import tpu_sc as plsc

N = 10000
E = 320000
D_IN = 128
D_OUT = 2 * D_IN

NC = 2
NS = 16
EPT = E // NS
CH = 80
NITER = EPT // CH
CW = 16
L = 16

N_PAD = 10240
HALF = N_PAD // NC
ACC_ROWS = HALF + 8
ROWS_PT = HALF // NS
DR = 80
NDR = ROWS_PT // DR

_mesh = plsc.VectorSubcoreMesh(core_axis_name="c", subcore_axis_name="s")


@functools.partial(
    pl.kernel,
    mesh=_mesh,
    out_type=jax.ShapeDtypeStruct((N_PAD, D_IN), jnp.float32),
    scratch_types=[
        pltpu.VMEM((CH,), jnp.int32),
        pltpu.VMEM((CH,), jnp.int32),
        pltpu.VMEM((CH, D_IN), jnp.float32),
        pltpu.VMEM((CH, CW), jnp.float32),
        pltpu.VMEM((DR, D_IN), jnp.float32),
        pltpu.VMEM((DR, CW), jnp.float32),
        pltpu.VMEM_SHARED((ACC_ROWS, D_IN), jnp.float32),
        pltpu.VMEM_SHARED((ACC_ROWS, CW), jnp.float32),
        pltpu.SemaphoreType.DMA,
    ],
)
def _sc_aggregate(src_hbm, dst_hbm, x_hbm, zrows_hbm,
                  out_hbm,
                  src_v, dst_v, rows_v, ones_v, drain_v, cdrain_v,
                  acc_sh, cnt_sh, sem):
    c = lax.axis_index("c")
    s = lax.axis_index("s")
    row0 = s * ROWS_PT
    lo = c * HALF

    one16 = jnp.ones((L,), jnp.float32)
    zero16 = jnp.zeros((L,), jnp.float32)

    for r in range(CH):
        ones_v[r, :] = one16
    for r in range(DR):
        cdrain_v[r, :] = zero16

    pltpu.sync_copy(zrows_hbm, drain_v)
    for k in range(NDR):
        pltpu.sync_copy(drain_v, acc_sh.at[pl.ds(row0 + k * DR, DR)])
        pltpu.sync_copy(cdrain_v, cnt_sh.at[pl.ds(row0 + k * DR, DR)])
    plsc.subcore_barrier()

    def body(i, carry):
        base = s * EPT + i * CH
        pltpu.sync_copy(src_hbm.at[pl.ds(base, CH)], src_v)
        pltpu.sync_copy(dst_hbm.at[pl.ds(base, CH)], dst_v)
        pltpu.async_copy(x_hbm.at[src_v], rows_v, sem).wait()
        for j in range(CH // L):
            d = dst_v[pl.ds(j * L, L)]
            t = d - lo
            ok = (t >= 0) & (t < HALF)
            dst_v[pl.ds(j * L, L)] = jnp.where(ok, t, HALF)
        pltpu.sync_copy(rows_v, acc_sh.at[dst_v], add=True)
        pltpu.sync_copy(ones_v, cnt_sh.at[dst_v], add=True)
        return carry

    lax.fori_loop(0, NITER, body, 0)
    plsc.subcore_barrier()

    out_base = c * HALF + row0
    for k in range(NDR):
        pltpu.sync_copy(acc_sh.at[pl.ds(row0 + k * DR, DR)], drain_v)
        pltpu.sync_copy(cnt_sh.at[pl.ds(row0 + k * DR, DR)], cdrain_v)

        def div_row(r, carry):
            inv = 1.0 / jnp.maximum(cdrain_v[r, :], 1.0)
            for j in range(D_IN // L):
                drain_v[r, pl.ds(j * L, L)] = drain_v[r, pl.ds(j * L, L)] * inv
            return carry

        lax.fori_loop(0, DR, div_row, 0)
        pltpu.sync_copy(drain_v, out_hbm.at[pl.ds(out_base + k * DR, DR)])


_TB = 640
_NB = N_PAD // _TB


def _tc_update(agg_ref, x_ref, wl_ref, bl_ref, wr_ref, out_ref):
    h = (jnp.dot(agg_ref[...], wl_ref[...], preferred_element_type=jnp.float32)
         + bl_ref[...]
         + jnp.dot(x_ref[...], wr_ref[...], preferred_element_type=jnp.float32))
    out_ref[...] = 0.5 * h * (1.0 + lax.erf(h * 0.7071067811865476))


def kernel(x, edge_index, W_l, b_l, W_r):
    src = edge_index[0]
    dst = edge_index[1]
    zrows = jnp.zeros((DR, D_IN), dtype=jnp.float32)

    agg = _sc_aggregate(src, dst, x, zrows)

    x_pad = jnp.pad(x, ((0, N_PAD - N), (0, 0)))
    h = pl.pallas_call(
        _tc_update,
        grid=(_NB,),
        in_specs=[
            pl.BlockSpec((_TB, D_IN), lambda i: (i, 0)),
            pl.BlockSpec((_TB, D_IN), lambda i: (i, 0)),
            pl.BlockSpec((D_IN, D_OUT), lambda i: (0, 0)),
            pl.BlockSpec((1, D_OUT), lambda i: (0, 0)),
            pl.BlockSpec((D_IN, D_OUT), lambda i: (0, 0)),
        ],
        out_specs=pl.BlockSpec((_TB, D_OUT), lambda i: (i, 0)),
        out_shape=jax.ShapeDtypeStruct((N_PAD, D_OUT), jnp.float32),
    )(agg, x_pad, W_l, b_l.reshape(1, D_OUT), W_r)
    return h[:N].reshape(-1)
